# baseline (speedup 1.0000x reference)
import jax
import jax.numpy as jnp
from jax.experimental import pallas as pl
from jax.experimental.pallas import tpu as pltpu

_NCHUNK = 4


def _copy_body(x_ref, o_ref, *sems):
    n = x_ref.shape[0]
    chunk = n // _NCHUNK
    for i in range(_NCHUNK):
        pltpu.make_async_copy(
            x_ref.at[pl.ds(i * chunk, chunk)],
            o_ref.at[pl.ds(i * chunk, chunk)],
            sems[i],
        ).start()
    for i in range(_NCHUNK):
        pltpu.make_async_copy(
            x_ref.at[pl.ds(i * chunk, chunk)],
            o_ref.at[pl.ds(i * chunk, chunk)],
            sems[i],
        ).wait()


def kernel(x):
    return pl.pallas_call(
        _copy_body,
        out_shape=jax.ShapeDtypeStruct(x.shape, x.dtype),
        in_specs=[pl.BlockSpec(memory_space=pltpu.MemorySpace.HBM)],
        out_specs=pl.BlockSpec(memory_space=pltpu.MemorySpace.HBM),
        scratch_shapes=[pltpu.SemaphoreType.DMA] * _NCHUNK,
    )(x)


# speedup vs baseline: 12.3216x; 12.3216x over previous
import jax
import jax.numpy as jnp
from jax.experimental import pallas as pl

_BLOCK_N = 2


def _copy_body(x_ref, o_ref):
    o_ref[...] = x_ref[...]


def kernel(x):
    n, c, h, w = x.shape
    return pl.pallas_call(
        _copy_body,
        out_shape=jax.ShapeDtypeStruct(x.shape, x.dtype),
        grid=(n // _BLOCK_N,),
        in_specs=[pl.BlockSpec((_BLOCK_N, c, h, w), lambda i: (i, 0, 0, 0))],
        out_specs=pl.BlockSpec((_BLOCK_N, c, h, w), lambda i: (i, 0, 0, 0)),
    )(x)


# speedup vs baseline: 13.2742x; 1.0773x over previous
import jax
import jax.numpy as jnp
from jax.experimental import pallas as pl
from jax.experimental.pallas import tpu as pltpu

_BN = 4
_NBUF = 8
_LOOKAHEAD = 4


def _copy_body(x_hbm, o_hbm, buf, in_sems, out_sems):
    n = x_hbm.shape[0]
    chunks = n // _BN

    def in_copy(i):
        s = i % _NBUF
        return pltpu.make_async_copy(
            x_hbm.at[pl.ds(i * _BN, _BN)], buf.at[s], in_sems.at[s])

    def out_copy(i):
        s = i % _NBUF
        return pltpu.make_async_copy(
            buf.at[s], o_hbm.at[pl.ds(i * _BN, _BN)], out_sems.at[s])

    for i in range(-_LOOKAHEAD, chunks):
        j = i + _LOOKAHEAD
        if j < chunks:
            if j >= _NBUF:
                out_copy(j - _NBUF).wait()
            in_copy(j).start()
        if i >= 0:
            in_copy(i).wait()
            out_copy(i).start()
    for i in range(max(chunks - _NBUF, 0), chunks):
        out_copy(i).wait()


def kernel(x):
    n, c, h, w = x.shape
    return pl.pallas_call(
        _copy_body,
        out_shape=jax.ShapeDtypeStruct(x.shape, x.dtype),
        in_specs=[pl.BlockSpec(memory_space=pltpu.MemorySpace.HBM)],
        out_specs=pl.BlockSpec(memory_space=pltpu.MemorySpace.HBM),
        scratch_shapes=[
            pltpu.VMEM((_NBUF, _BN, c, h, w), x.dtype),
            pltpu.SemaphoreType.DMA((_NBUF,)),
            pltpu.SemaphoreType.DMA((_NBUF,)),
        ],
    )(x)
